# baseline (speedup 1.0000x reference)
import functools

import jax
import jax.numpy as jnp
from jax.experimental import pallas as pl
from jax.experimental.pallas import tpu as pltpu

NUM_LEVELS = 4
STEP_RATIO = 2
KNN = 16
FM_KNN = 3

F32 = jnp.float32
HI = jax.lax.Precision.HIGHEST
BIG = 1e30
IBIG = 2 ** 30


def _dot(a, b, dims):
    return jax.lax.dot_general(a, b, (dims, ((), ())), precision=HI,
                               preferred_element_type=F32)


def _cp(ndims):
    return pltpu.CompilerParams(dimension_semantics=("parallel",) * ndims)


def _cdiv(a, b):
    return (a + b - 1) // b


def _block_t(n):
    return min(256, _cdiv(n, 128) * 128)


def _norm_l0_body(xyz_ref, w_ref, b_ref, xn_ref, cent_ref, rad_ref, x0_ref):
    pc = xyz_ref[0]
    cent = jnp.mean(pc, axis=1, keepdims=True)
    pcc = pc - cent
    rad = jnp.max(jnp.sqrt(jnp.sum(pcc * pcc, axis=0, keepdims=True)),
                  axis=1, keepdims=True)
    xn = pcc / rad
    xn_ref[0] = xn
    cent_ref[0] = cent
    rad_ref[0] = rad
    x0_ref[0] = _dot(w_ref[...], xn, ((1,), (0,))) + b_ref[...]


def _norm_l0(xyz, w, b):
    B, _, N = xyz.shape
    return pl.pallas_call(
        _norm_l0_body,
        grid=(B,),
        compiler_params=_cp(1),
        in_specs=[
            pl.BlockSpec((1, 3, N), lambda i: (i, 0, 0)),
            pl.BlockSpec((24, 3), lambda i: (0, 0)),
            pl.BlockSpec((24, 1), lambda i: (0, 0)),
        ],
        out_specs=[
            pl.BlockSpec((1, 3, N), lambda i: (i, 0, 0)),
            pl.BlockSpec((1, 3, 1), lambda i: (i, 0, 0)),
            pl.BlockSpec((1, 1, 1), lambda i: (i, 0, 0)),
            pl.BlockSpec((1, 24, N), lambda i: (i, 0, 0)),
        ],
        out_shape=[
            jax.ShapeDtypeStruct((B, 3, N), F32),
            jax.ShapeDtypeStruct((B, 3, 1), F32),
            jax.ShapeDtypeStruct((B, 1, 1), F32),
            jax.ShapeDtypeStruct((B, 24, N), F32),
        ],
    )(xyz, w, b.reshape(24, 1))


def _ec_body(x_ref, xr_ref, wu_ref, wv_ref, w1a_ref, w1b_ref, w2a_ref,
             w2b_ref, w2c_ref, b0_ref, b1_ref, b2_ref, y_ref, *, T):
    x = x_ref[0]
    xr = xr_ref[0]
    N = x.shape[1]
    base = pl.program_id(1) * T

    pp = jnp.sum(x * x, axis=0, keepdims=True)
    d = pp - 2.0 * _dot(xr, x, ((0,), (0,)))

    lane = jax.lax.broadcasted_iota(jnp.int32, (T, N), 1)
    row = base + jax.lax.broadcasted_iota(jnp.int32, (T, N), 0)
    d = jnp.where(lane == row, BIG, d)

    v0t = _dot(x, wv_ref[...], ((0,), (1,)))
    u0 = _dot(xr, wu_ref[...], ((0,), (1,))) + b0_ref[...]
    w1x = _dot(xr, w1b_ref[...], ((0,), (1,))) + b1_ref[...]
    w2x = _dot(xr, w2c_ref[...], ((0,), (1,))) + b2_ref[...]

    w1a = w1a_ref[...]
    w2a = w2a_ref[...]
    w2b = w2b_ref[...]

    neg = jnp.float32(-BIG)
    c2a = jnp.full((T, 12), neg, F32)
    r1a = jnp.full((T, 12), neg, F32)
    r0a = jnp.full((T, 12), neg, F32)

    for _ in range(KNN):
        m = jnp.min(d, axis=1, keepdims=True)
        cand = jnp.where(d <= m, lane, IBIG)
        idx = jnp.min(cand, axis=1, keepdims=True)
        onehot = cand == idx
        sel = onehot.astype(F32)
        g = _dot(sel, v0t, ((1,), (0,)))
        d = jnp.where(onehot, BIG, d)
        r0 = jax.nn.relu(u0 + g)
        r1 = jax.nn.relu(_dot(r0, w1a, ((1,), (1,))) + w1x)
        c2 = _dot(r1, w2a, ((1,), (1,))) + _dot(r0, w2b, ((1,), (1,))) + w2x
        c2a = jnp.maximum(c2a, c2)
        r1a = jnp.maximum(r1a, r1)
        r0a = jnp.maximum(r0a, r0)

    y = jnp.concatenate([c2a, r1a, r0a], axis=1)
    y_ref[0] = y.T


def _edge_conv(x, p):
    B, _, N = x.shape
    T = _block_t(N)
    nb = _cdiv(N, T)
    w0, w1, w2 = p['W0'], p['W1'], p['W2']
    wu = w0[:, :24] - w0[:, 24:]
    wv = w0[:, 24:]
    args = (x, x, wu, wv, w1[:, :12], w1[:, 12:], w2[:, :12], w2[:, 12:24],
            w2[:, 24:], p['b0'].reshape(1, 12), p['b1'].reshape(1, 12),
            p['b2'].reshape(1, 12))
    wspec = lambda s: pl.BlockSpec(s, lambda b, i: (0,) * len(s))
    return pl.pallas_call(
        functools.partial(_ec_body, T=T),
        grid=(B, nb),
        compiler_params=_cp(2),
        in_specs=[
            pl.BlockSpec((1, 24, N), lambda b, i: (b, 0, 0)),
            pl.BlockSpec((1, 24, T), lambda b, i: (b, 0, i)),
            wspec((12, 24)), wspec((12, 24)), wspec((12, 12)),
            wspec((12, 24)), wspec((12, 12)), wspec((12, 12)),
            wspec((12, 24)), wspec((1, 12)), wspec((1, 12)), wspec((1, 12)),
        ],
        out_specs=pl.BlockSpec((1, 36, T), lambda b, i: (b, 0, i)),
        out_shape=jax.ShapeDtypeStruct((B, 36, N), F32),
    )(*args)


def _proj_body(*refs, nparts):
    parts = refs[:nparts]
    ws = refs[nparts:2 * nparts]
    b_ref = refs[2 * nparts]
    out_ref = refs[2 * nparts + 1]
    acc = b_ref[...].T
    for pr, wr in zip(parts, ws):
        acc = acc + _dot(wr[...], pr[0], ((1,), (0,)))
    out_ref[0] = jax.nn.relu(acc)


def _proj(parts, ws, b):
    B, _, N = parts[0].shape
    T = _block_t(N)
    nb = _cdiv(N, T)
    in_specs = [pl.BlockSpec((1, p.shape[1], T), lambda b_, i: (b_, 0, i))
                for p in parts]
    in_specs += [pl.BlockSpec(w.shape, lambda b_, i: (0, 0)) for w in ws]
    in_specs += [pl.BlockSpec((1, 24), lambda b_, i: (0, 0))]
    return pl.pallas_call(
        functools.partial(_proj_body, nparts=len(parts)),
        grid=(B, nb),
        compiler_params=_cp(2),
        in_specs=in_specs,
        out_specs=pl.BlockSpec((1, 24, T), lambda b_, i: (b_, 0, i)),
        out_shape=jax.ShapeDtypeStruct((B, 24, N), F32),
    )(*parts, *ws, b.reshape(1, 24))


def _fuse_knn_body(q_ref, qr_ref, pxyz_ref, pfeat_ref, x_ref,
                   sd_ref, fd_ref, feats_ref, *, T):
    qr = qr_ref[0]
    pxyz = pxyz_ref[0]
    pfeat = pfeat_ref[0]
    M = pxyz.shape[1]
    del q_ref

    pp = jnp.sum(pxyz * pxyz, axis=0, keepdims=True)
    d = pp - 2.0 * _dot(qr, pxyz, ((0,), (0,)))
    lane = jax.lax.broadcasted_iota(jnp.int32, (T, M), 1)

    qt = qr.T
    xt = x_ref[0].T

    sds, fds = [], []
    for k in range(FM_KNN):
        m = jnp.min(d, axis=1, keepdims=True)
        cand = jnp.where(d <= m, lane, IBIG)
        idx = jnp.min(cand, axis=1, keepdims=True)
        onehot = cand == idx
        sel = onehot.astype(F32)
        d = jnp.where(onehot, BIG, d)
        feat = _dot(sel, pfeat, ((1,), (1,)))
        pt = _dot(sel, pxyz, ((1,), (1,)))
        sds.append(jnp.sum((qt - pt) ** 2, axis=1, keepdims=True))
        fds.append(jnp.sum((xt - feat) ** 2, axis=1, keepdims=True))
        feats_ref[0, k] = feat
    sd_ref[0] = jnp.concatenate(sds, axis=1)
    fd_ref[0] = jnp.concatenate(fds, axis=1)


def _fuse_h_body(sd_ref, fd_ref, hs_ref, hf_ref):
    hs_ref[0] = jnp.mean(jnp.min(sd_ref[0], axis=1, keepdims=True),
                         axis=0, keepdims=True)
    hf_ref[0] = jnp.mean(jnp.min(fd_ref[0], axis=1, keepdims=True),
                         axis=0, keepdims=True)


def _fuse_apply_body(x_ref, feats_ref, sd_ref, fd_ref, hs_ref, hf_ref,
                     out_ref):
    hs = hs_ref[0]
    hf = hf_ref[0]
    sw = jnp.exp(-sd_ref[0] / (hs * 0.5))
    fw = jnp.exp(-fd_ref[0] / (hf * 0.5))
    w = sw * fw
    w = w / jnp.sum(w + 1e-05, axis=1, keepdims=True)
    corr = (w[:, 0:1] * feats_ref[0, 0] + w[:, 1:2] * feats_ref[0, 1]
            + w[:, 2:3] * feats_ref[0, 2])
    out_ref[0] = x_ref[0] + 0.2 * corr.T


def _fusion(xyz, prev_xyz, prev_feat, x):
    B, _, N = xyz.shape
    M = prev_xyz.shape[2]
    C = x.shape[1]
    T = _block_t(N)
    nb = _cdiv(N, T)
    sd, fd, feats = pl.pallas_call(
        functools.partial(_fuse_knn_body, T=T),
        grid=(B, nb),
        compiler_params=_cp(2),
        in_specs=[
            pl.BlockSpec((1, 3, N), lambda b, i: (b, 0, 0)),
            pl.BlockSpec((1, 3, T), lambda b, i: (b, 0, i)),
            pl.BlockSpec((1, 3, M), lambda b, i: (b, 0, 0)),
            pl.BlockSpec((1, C, M), lambda b, i: (b, 0, 0)),
            pl.BlockSpec((1, C, T), lambda b, i: (b, 0, i)),
        ],
        out_specs=[
            pl.BlockSpec((1, T, FM_KNN), lambda b, i: (b, i, 0)),
            pl.BlockSpec((1, T, FM_KNN), lambda b, i: (b, i, 0)),
            pl.BlockSpec((1, FM_KNN, T, C), lambda b, i: (b, 0, i, 0)),
        ],
        out_shape=[
            jax.ShapeDtypeStruct((B, N, FM_KNN), F32),
            jax.ShapeDtypeStruct((B, N, FM_KNN), F32),
            jax.ShapeDtypeStruct((B, FM_KNN, N, C), F32),
        ],
    )(xyz, xyz, prev_xyz, prev_feat, x)

    hs, hf = pl.pallas_call(
        _fuse_h_body,
        grid=(B,),
        compiler_params=_cp(1),
        in_specs=[
            pl.BlockSpec((1, N, FM_KNN), lambda b: (b, 0, 0)),
            pl.BlockSpec((1, N, FM_KNN), lambda b: (b, 0, 0)),
        ],
        out_specs=[
            pl.BlockSpec((1, 1, 1), lambda b: (b, 0, 0)),
            pl.BlockSpec((1, 1, 1), lambda b: (b, 0, 0)),
        ],
        out_shape=[
            jax.ShapeDtypeStruct((B, 1, 1), F32),
            jax.ShapeDtypeStruct((B, 1, 1), F32),
        ],
    )(sd, fd)

    return pl.pallas_call(
        _fuse_apply_body,
        grid=(B, nb),
        compiler_params=_cp(2),
        in_specs=[
            pl.BlockSpec((1, C, T), lambda b, i: (b, 0, i)),
            pl.BlockSpec((1, FM_KNN, T, C), lambda b, i: (b, 0, i, 0)),
            pl.BlockSpec((1, T, FM_KNN), lambda b, i: (b, i, 0)),
            pl.BlockSpec((1, T, FM_KNN), lambda b, i: (b, i, 0)),
            pl.BlockSpec((1, 1, 1), lambda b, i: (b, 0, 0)),
            pl.BlockSpec((1, 1, 1), lambda b, i: (b, 0, 0)),
        ],
        out_specs=pl.BlockSpec((1, C, T), lambda b, i: (b, 0, i)),
        out_shape=jax.ShapeDtypeStruct((B, C, N), F32),
    )(x, feats, sd, fd, hs, hf)


def _ups_body(x_ref, xn_ref, cent_ref, rad_ref, w1_ref, bia_ref, bib_ref,
              w2_ref, b2_ref, w3_ref, b3_ref, w4_ref, b4_ref,
              o0_ref, o1_ref):
    x = x_ref[0]
    t = _dot(w1_ref[...], x, ((1,), (0,)))
    cent = cent_ref[0]
    rad = rad_ref[0]
    base = xn_ref[0]
    for bias_ref, o_ref in ((bia_ref, o0_ref), (bib_ref, o1_ref)):
        u = jax.nn.relu(t + bias_ref[...])
        u = jax.nn.relu(_dot(w2_ref[...], u, ((1,), (0,))) + b2_ref[...])
        u = jax.nn.relu(_dot(w3_ref[...], u, ((1,), (0,))) + b3_ref[...])
        u = _dot(w4_ref[...], u, ((1,), (0,))) + b4_ref[...]
        o_ref[0] = (u + base) * rad + cent


def _upsample(x, xyz_norm, cent, rad, p):
    B, C, N = x.shape
    T = _block_t(N)
    nb = _cdiv(N, T)
    w1 = p['up1_W'][:, :264]
    wc = p['up1_W'][:, 264:]
    code = jnp.linspace(-0.2, 0.2, STEP_RATIO, dtype=F32)
    bia = (p['up1_b'].reshape(128, 1) + wc * code[0])
    bib = (p['up1_b'].reshape(128, 1) + wc * code[1])
    wspec = lambda s: pl.BlockSpec(s, lambda b, i: (0,) * len(s))
    o0, o1 = pl.pallas_call(
        _ups_body,
        grid=(B, nb),
        compiler_params=_cp(2),
        in_specs=[
            pl.BlockSpec((1, C, T), lambda b, i: (b, 0, i)),
            pl.BlockSpec((1, 3, T), lambda b, i: (b, 0, i)),
            pl.BlockSpec((1, 3, 1), lambda b, i: (b, 0, 0)),
            pl.BlockSpec((1, 1, 1), lambda b, i: (b, 0, 0)),
            wspec((128, 264)), wspec((128, 1)), wspec((128, 1)),
            wspec((128, 128)), wspec((128, 1)),
            wspec((64, 128)), wspec((64, 1)),
            wspec((3, 64)), wspec((3, 1)),
        ],
        out_specs=[
            pl.BlockSpec((1, 3, T), lambda b, i: (b, 0, i)),
            pl.BlockSpec((1, 3, T), lambda b, i: (b, 0, i)),
        ],
        out_shape=[
            jax.ShapeDtypeStruct((B, 3, N), F32),
            jax.ShapeDtypeStruct((B, 3, N), F32),
        ],
    )(x, xyz_norm, cent, rad, w1, bia, bib,
      p['up2_W'], p['up2_b'].reshape(128, 1),
      p['fc1_W'], p['fc1_b'].reshape(64, 1),
      p['fc2_W'], p['fc2_b'].reshape(3, 1))
    return jnp.stack([o0, o1], axis=-1).reshape(B, 3, N * STEP_RATIO)


def _level(xyz, prev, p):
    xyz_norm, cent, rad, x0 = _norm_l0(xyz, p['l0_W'], p['l0_b'])
    e1 = _edge_conv(x0, p['ec1'])
    w = p['p2_W']
    xp2 = _proj([e1, x0], [w[:, :36], w[:, 36:60] + w[:, 60:84]], p['p2_b'])
    e2 = _edge_conv(xp2, p['ec2'])
    w = p['p3_W']
    xp3 = _proj([e2, xp2, e1, x0],
                [w[:, :36], w[:, 36:60], w[:, 60:96], w[:, 96:120] + w[:, 120:144]],
                p['p3_b'])
    e3 = _edge_conv(xp3, p['ec3'])
    w = p['p4_W']
    xp4 = _proj([e3, xp3, e2, xp2, e1, x0],
                [w[:, :36], w[:, 36:60], w[:, 60:96], w[:, 96:120],
                 w[:, 120:156], w[:, 156:180] + w[:, 180:204]],
                p['p4_b'])
    e4 = _edge_conv(xp4, p['ec4'])
    x = jnp.concatenate([e4, xp4, e3, xp3, e2, xp2, e1, x0, x0], axis=1)
    if prev is not None:
        prev_xyz, prev_feat = prev
        x = _fusion(xyz, prev_xyz, prev_feat, x)
    new_xyz = _upsample(x, xyz_norm, cent, rad, p)
    return new_xyz, x


def kernel(xyz, params):
    prev = None
    for l in range(1, NUM_LEVELS + 1):
        new_xyz, feats = _level(xyz, prev, params['level_%d' % l])
        prev = (xyz, feats)
        xyz = new_xyz
    return xyz


# speedup vs baseline: 1.0262x; 1.0262x over previous
import functools

import jax
import jax.numpy as jnp
from jax.experimental import pallas as pl
from jax.experimental.pallas import tpu as pltpu

NUM_LEVELS = 4
STEP_RATIO = 2
KNN = 16
FM_KNN = 3

F32 = jnp.float32
HI = jax.lax.Precision.HIGHEST
BIG = 1e30
IBIG = 2 ** 30


def _dot(a, b, dims):
    return jax.lax.dot_general(a, b, (dims, ((), ())), precision=HI,
                               preferred_element_type=F32)


def _cp(ndims):
    return pltpu.CompilerParams(dimension_semantics=("parallel",) * ndims)


def _cdiv(a, b):
    return (a + b - 1) // b


def _block_t(n):
    return min(256, _cdiv(n, 128) * 128)


def _norm_l0_body(xyz_ref, w_ref, b_ref, xn_ref, cent_ref, rad_ref, x0_ref):
    pc = xyz_ref[0]
    cent = jnp.mean(pc, axis=1, keepdims=True)
    pcc = pc - cent
    rad = jnp.max(jnp.sqrt(jnp.sum(pcc * pcc, axis=0, keepdims=True)),
                  axis=1, keepdims=True)
    xn = pcc / rad
    xn_ref[0] = xn
    cent_ref[0] = cent
    rad_ref[0] = rad
    x0_ref[0] = _dot(w_ref[...], xn, ((1,), (0,))) + b_ref[...]


def _norm_l0(xyz, w, b):
    B, _, N = xyz.shape
    return pl.pallas_call(
        _norm_l0_body,
        grid=(B,),
        compiler_params=_cp(1),
        in_specs=[
            pl.BlockSpec((1, 3, N), lambda i: (i, 0, 0)),
            pl.BlockSpec((24, 3), lambda i: (0, 0)),
            pl.BlockSpec((24, 1), lambda i: (0, 0)),
        ],
        out_specs=[
            pl.BlockSpec((1, 3, N), lambda i: (i, 0, 0)),
            pl.BlockSpec((1, 3, 1), lambda i: (i, 0, 0)),
            pl.BlockSpec((1, 1, 1), lambda i: (i, 0, 0)),
            pl.BlockSpec((1, 24, N), lambda i: (i, 0, 0)),
        ],
        out_shape=[
            jax.ShapeDtypeStruct((B, 3, N), F32),
            jax.ShapeDtypeStruct((B, 3, 1), F32),
            jax.ShapeDtypeStruct((B, 1, 1), F32),
            jax.ShapeDtypeStruct((B, 24, N), F32),
        ],
    )(xyz, w, b.reshape(24, 1))


def _ec_body(x_ref, xr_ref, wu_ref, wv_ref, w1a_ref, w1b_ref, w2a_ref,
             w2b_ref, w2c_ref, b0_ref, b1_ref, b2_ref, y_ref, *, T):
    x = x_ref[0]
    xr = xr_ref[0]
    N = x.shape[1]
    base = pl.program_id(1) * T

    pp = jnp.sum(x * x, axis=0, keepdims=True)
    d = pp - 2.0 * _dot(xr, x, ((0,), (0,)))

    lane = jax.lax.broadcasted_iota(jnp.int32, (T, N), 1)
    row = base + jax.lax.broadcasted_iota(jnp.int32, (T, N), 0)
    d = jnp.where(lane == row, BIG, d)

    v0t = _dot(x, wv_ref[...], ((0,), (1,)))
    u0 = _dot(xr, wu_ref[...], ((0,), (1,))) + b0_ref[...]
    w1x = _dot(xr, w1b_ref[...], ((0,), (1,))) + b1_ref[...]
    w2x = _dot(xr, w2c_ref[...], ((0,), (1,))) + b2_ref[...]

    w1a = w1a_ref[...]
    w2a = w2a_ref[...]
    w2b = w2b_ref[...]

    neg = jnp.float32(-BIG)
    c2a = jnp.full((T, 12), neg, F32)
    r1a = jnp.full((T, 12), neg, F32)
    r0a = jnp.full((T, 12), neg, F32)

    for _ in range(KNN):
        idx = jnp.argmin(d, axis=1).astype(jnp.int32).reshape(T, 1)
        onehot = lane == idx
        sel = onehot.astype(F32)
        g = _dot(sel, v0t, ((1,), (0,)))
        d = jnp.where(onehot, BIG, d)
        r0 = jax.nn.relu(u0 + g)
        r1 = jax.nn.relu(_dot(r0, w1a, ((1,), (1,))) + w1x)
        c2 = _dot(r1, w2a, ((1,), (1,))) + _dot(r0, w2b, ((1,), (1,))) + w2x
        c2a = jnp.maximum(c2a, c2)
        r1a = jnp.maximum(r1a, r1)
        r0a = jnp.maximum(r0a, r0)

    y = jnp.concatenate([c2a, r1a, r0a], axis=1)
    y_ref[0] = y.T


def _edge_conv(x, p):
    B, _, N = x.shape
    T = _block_t(N)
    nb = _cdiv(N, T)
    w0, w1, w2 = p['W0'], p['W1'], p['W2']
    wu = w0[:, :24] - w0[:, 24:]
    wv = w0[:, 24:]
    args = (x, x, wu, wv, w1[:, :12], w1[:, 12:], w2[:, :12], w2[:, 12:24],
            w2[:, 24:], p['b0'].reshape(1, 12), p['b1'].reshape(1, 12),
            p['b2'].reshape(1, 12))
    wspec = lambda s: pl.BlockSpec(s, lambda b, i: (0,) * len(s))
    return pl.pallas_call(
        functools.partial(_ec_body, T=T),
        grid=(B, nb),
        compiler_params=_cp(2),
        in_specs=[
            pl.BlockSpec((1, 24, N), lambda b, i: (b, 0, 0)),
            pl.BlockSpec((1, 24, T), lambda b, i: (b, 0, i)),
            wspec((12, 24)), wspec((12, 24)), wspec((12, 12)),
            wspec((12, 24)), wspec((12, 12)), wspec((12, 12)),
            wspec((12, 24)), wspec((1, 12)), wspec((1, 12)), wspec((1, 12)),
        ],
        out_specs=pl.BlockSpec((1, 36, T), lambda b, i: (b, 0, i)),
        out_shape=jax.ShapeDtypeStruct((B, 36, N), F32),
    )(*args)


def _proj_body(*refs, nparts):
    parts = refs[:nparts]
    ws = refs[nparts:2 * nparts]
    b_ref = refs[2 * nparts]
    out_ref = refs[2 * nparts + 1]
    acc = b_ref[...].T
    for pr, wr in zip(parts, ws):
        acc = acc + _dot(wr[...], pr[0], ((1,), (0,)))
    out_ref[0] = jax.nn.relu(acc)


def _proj(parts, ws, b):
    B, _, N = parts[0].shape
    T = _block_t(N)
    nb = _cdiv(N, T)
    in_specs = [pl.BlockSpec((1, p.shape[1], T), lambda b_, i: (b_, 0, i))
                for p in parts]
    in_specs += [pl.BlockSpec(w.shape, lambda b_, i: (0, 0)) for w in ws]
    in_specs += [pl.BlockSpec((1, 24), lambda b_, i: (0, 0))]
    return pl.pallas_call(
        functools.partial(_proj_body, nparts=len(parts)),
        grid=(B, nb),
        compiler_params=_cp(2),
        in_specs=in_specs,
        out_specs=pl.BlockSpec((1, 24, T), lambda b_, i: (b_, 0, i)),
        out_shape=jax.ShapeDtypeStruct((B, 24, N), F32),
    )(*parts, *ws, b.reshape(1, 24))


def _fuse_knn_body(q_ref, qr_ref, pxyz_ref, pfeat_ref, x_ref,
                   sd_ref, fd_ref, feats_ref, *, T):
    qr = qr_ref[0]
    pxyz = pxyz_ref[0]
    pfeat = pfeat_ref[0]
    M = pxyz.shape[1]
    del q_ref

    pp = jnp.sum(pxyz * pxyz, axis=0, keepdims=True)
    d = pp - 2.0 * _dot(qr, pxyz, ((0,), (0,)))
    lane = jax.lax.broadcasted_iota(jnp.int32, (T, M), 1)

    qt = qr.T
    xt = x_ref[0].T

    sds, fds = [], []
    for k in range(FM_KNN):
        idx = jnp.argmin(d, axis=1).astype(jnp.int32).reshape(T, 1)
        onehot = lane == idx
        sel = onehot.astype(F32)
        d = jnp.where(onehot, BIG, d)
        feat = _dot(sel, pfeat, ((1,), (1,)))
        pt = _dot(sel, pxyz, ((1,), (1,)))
        sds.append(jnp.sum((qt - pt) ** 2, axis=1, keepdims=True))
        fds.append(jnp.sum((xt - feat) ** 2, axis=1, keepdims=True))
        feats_ref[0, k] = feat
    sd_ref[0] = jnp.concatenate(sds, axis=1)
    fd_ref[0] = jnp.concatenate(fds, axis=1)


def _fuse_h_body(sd_ref, fd_ref, hs_ref, hf_ref):
    hs_ref[0] = jnp.mean(jnp.min(sd_ref[0], axis=1, keepdims=True),
                         axis=0, keepdims=True)
    hf_ref[0] = jnp.mean(jnp.min(fd_ref[0], axis=1, keepdims=True),
                         axis=0, keepdims=True)


def _fuse_apply_body(x_ref, feats_ref, sd_ref, fd_ref, hs_ref, hf_ref,
                     out_ref):
    hs = hs_ref[0]
    hf = hf_ref[0]
    sw = jnp.exp(-sd_ref[0] / (hs * 0.5))
    fw = jnp.exp(-fd_ref[0] / (hf * 0.5))
    w = sw * fw
    w = w / jnp.sum(w + 1e-05, axis=1, keepdims=True)
    corr = (w[:, 0:1] * feats_ref[0, 0] + w[:, 1:2] * feats_ref[0, 1]
            + w[:, 2:3] * feats_ref[0, 2])
    out_ref[0] = x_ref[0] + 0.2 * corr.T


def _fusion(xyz, prev_xyz, prev_feat, x):
    B, _, N = xyz.shape
    M = prev_xyz.shape[2]
    C = x.shape[1]
    T = _block_t(N)
    nb = _cdiv(N, T)
    sd, fd, feats = pl.pallas_call(
        functools.partial(_fuse_knn_body, T=T),
        grid=(B, nb),
        compiler_params=_cp(2),
        in_specs=[
            pl.BlockSpec((1, 3, N), lambda b, i: (b, 0, 0)),
            pl.BlockSpec((1, 3, T), lambda b, i: (b, 0, i)),
            pl.BlockSpec((1, 3, M), lambda b, i: (b, 0, 0)),
            pl.BlockSpec((1, C, M), lambda b, i: (b, 0, 0)),
            pl.BlockSpec((1, C, T), lambda b, i: (b, 0, i)),
        ],
        out_specs=[
            pl.BlockSpec((1, T, FM_KNN), lambda b, i: (b, i, 0)),
            pl.BlockSpec((1, T, FM_KNN), lambda b, i: (b, i, 0)),
            pl.BlockSpec((1, FM_KNN, T, C), lambda b, i: (b, 0, i, 0)),
        ],
        out_shape=[
            jax.ShapeDtypeStruct((B, N, FM_KNN), F32),
            jax.ShapeDtypeStruct((B, N, FM_KNN), F32),
            jax.ShapeDtypeStruct((B, FM_KNN, N, C), F32),
        ],
    )(xyz, xyz, prev_xyz, prev_feat, x)

    hs, hf = pl.pallas_call(
        _fuse_h_body,
        grid=(B,),
        compiler_params=_cp(1),
        in_specs=[
            pl.BlockSpec((1, N, FM_KNN), lambda b: (b, 0, 0)),
            pl.BlockSpec((1, N, FM_KNN), lambda b: (b, 0, 0)),
        ],
        out_specs=[
            pl.BlockSpec((1, 1, 1), lambda b: (b, 0, 0)),
            pl.BlockSpec((1, 1, 1), lambda b: (b, 0, 0)),
        ],
        out_shape=[
            jax.ShapeDtypeStruct((B, 1, 1), F32),
            jax.ShapeDtypeStruct((B, 1, 1), F32),
        ],
    )(sd, fd)

    return pl.pallas_call(
        _fuse_apply_body,
        grid=(B, nb),
        compiler_params=_cp(2),
        in_specs=[
            pl.BlockSpec((1, C, T), lambda b, i: (b, 0, i)),
            pl.BlockSpec((1, FM_KNN, T, C), lambda b, i: (b, 0, i, 0)),
            pl.BlockSpec((1, T, FM_KNN), lambda b, i: (b, i, 0)),
            pl.BlockSpec((1, T, FM_KNN), lambda b, i: (b, i, 0)),
            pl.BlockSpec((1, 1, 1), lambda b, i: (b, 0, 0)),
            pl.BlockSpec((1, 1, 1), lambda b, i: (b, 0, 0)),
        ],
        out_specs=pl.BlockSpec((1, C, T), lambda b, i: (b, 0, i)),
        out_shape=jax.ShapeDtypeStruct((B, C, N), F32),
    )(x, feats, sd, fd, hs, hf)


def _ups_body(x_ref, xn_ref, cent_ref, rad_ref, w1_ref, bia_ref, bib_ref,
              w2_ref, b2_ref, w3_ref, b3_ref, w4_ref, b4_ref,
              o0_ref, o1_ref):
    x = x_ref[0]
    t = _dot(w1_ref[...], x, ((1,), (0,)))
    cent = cent_ref[0]
    rad = rad_ref[0]
    base = xn_ref[0]
    for bias_ref, o_ref in ((bia_ref, o0_ref), (bib_ref, o1_ref)):
        u = jax.nn.relu(t + bias_ref[...])
        u = jax.nn.relu(_dot(w2_ref[...], u, ((1,), (0,))) + b2_ref[...])
        u = jax.nn.relu(_dot(w3_ref[...], u, ((1,), (0,))) + b3_ref[...])
        u = _dot(w4_ref[...], u, ((1,), (0,))) + b4_ref[...]
        o_ref[0] = (u + base) * rad + cent


def _upsample(x, xyz_norm, cent, rad, p):
    B, C, N = x.shape
    T = _block_t(N)
    nb = _cdiv(N, T)
    w1 = p['up1_W'][:, :264]
    wc = p['up1_W'][:, 264:]
    code = jnp.linspace(-0.2, 0.2, STEP_RATIO, dtype=F32)
    bia = (p['up1_b'].reshape(128, 1) + wc * code[0])
    bib = (p['up1_b'].reshape(128, 1) + wc * code[1])
    wspec = lambda s: pl.BlockSpec(s, lambda b, i: (0,) * len(s))
    o0, o1 = pl.pallas_call(
        _ups_body,
        grid=(B, nb),
        compiler_params=_cp(2),
        in_specs=[
            pl.BlockSpec((1, C, T), lambda b, i: (b, 0, i)),
            pl.BlockSpec((1, 3, T), lambda b, i: (b, 0, i)),
            pl.BlockSpec((1, 3, 1), lambda b, i: (b, 0, 0)),
            pl.BlockSpec((1, 1, 1), lambda b, i: (b, 0, 0)),
            wspec((128, 264)), wspec((128, 1)), wspec((128, 1)),
            wspec((128, 128)), wspec((128, 1)),
            wspec((64, 128)), wspec((64, 1)),
            wspec((3, 64)), wspec((3, 1)),
        ],
        out_specs=[
            pl.BlockSpec((1, 3, T), lambda b, i: (b, 0, i)),
            pl.BlockSpec((1, 3, T), lambda b, i: (b, 0, i)),
        ],
        out_shape=[
            jax.ShapeDtypeStruct((B, 3, N), F32),
            jax.ShapeDtypeStruct((B, 3, N), F32),
        ],
    )(x, xyz_norm, cent, rad, w1, bia, bib,
      p['up2_W'], p['up2_b'].reshape(128, 1),
      p['fc1_W'], p['fc1_b'].reshape(64, 1),
      p['fc2_W'], p['fc2_b'].reshape(3, 1))
    return jnp.stack([o0, o1], axis=-1).reshape(B, 3, N * STEP_RATIO)


def _level(xyz, prev, p):
    xyz_norm, cent, rad, x0 = _norm_l0(xyz, p['l0_W'], p['l0_b'])
    e1 = _edge_conv(x0, p['ec1'])
    w = p['p2_W']
    xp2 = _proj([e1, x0], [w[:, :36], w[:, 36:60] + w[:, 60:84]], p['p2_b'])
    e2 = _edge_conv(xp2, p['ec2'])
    w = p['p3_W']
    xp3 = _proj([e2, xp2, e1, x0],
                [w[:, :36], w[:, 36:60], w[:, 60:96], w[:, 96:120] + w[:, 120:144]],
                p['p3_b'])
    e3 = _edge_conv(xp3, p['ec3'])
    w = p['p4_W']
    xp4 = _proj([e3, xp3, e2, xp2, e1, x0],
                [w[:, :36], w[:, 36:60], w[:, 60:96], w[:, 96:120],
                 w[:, 120:156], w[:, 156:180] + w[:, 180:204]],
                p['p4_b'])
    e4 = _edge_conv(xp4, p['ec4'])
    x = jnp.concatenate([e4, xp4, e3, xp3, e2, xp2, e1, x0, x0], axis=1)
    if prev is not None:
        prev_xyz, prev_feat = prev
        x = _fusion(xyz, prev_xyz, prev_feat, x)
    new_xyz = _upsample(x, xyz_norm, cent, rad, p)
    return new_xyz, x


def kernel(xyz, params):
    prev = None
    for l in range(1, NUM_LEVELS + 1):
        new_xyz, feats = _level(xyz, prev, params['level_%d' % l])
        prev = (xyz, feats)
        xyz = new_xyz
    return xyz


# speedup vs baseline: 1.0405x; 1.0139x over previous
import functools

import jax
import jax.numpy as jnp
from jax.experimental import pallas as pl
from jax.experimental.pallas import tpu as pltpu

NUM_LEVELS = 4
STEP_RATIO = 2
KNN = 16
FM_KNN = 3

F32 = jnp.float32
HI = jax.lax.Precision.HIGHEST
BIG = 1e30
IBIG = 2 ** 30


def _dot(a, b, dims):
    return jax.lax.dot_general(a, b, (dims, ((), ())), precision=HI,
                               preferred_element_type=F32)


def _cp(ndims):
    return pltpu.CompilerParams(dimension_semantics=("parallel",) * ndims)


def _cdiv(a, b):
    return (a + b - 1) // b


def _block_t(n):
    return min(256, _cdiv(n, 128) * 128)


def _norm_l0_body(xyz_ref, w_ref, b_ref, xn_ref, cent_ref, rad_ref, x0_ref):
    pc = xyz_ref[0]
    cent = jnp.mean(pc, axis=1, keepdims=True)
    pcc = pc - cent
    rad = jnp.max(jnp.sqrt(jnp.sum(pcc * pcc, axis=0, keepdims=True)),
                  axis=1, keepdims=True)
    xn = pcc / rad
    xn_ref[0] = xn
    cent_ref[0] = cent
    rad_ref[0] = rad
    x0_ref[0] = _dot(w_ref[...], xn, ((1,), (0,))) + b_ref[...]


def _norm_l0(xyz, w, b):
    B, _, N = xyz.shape
    return pl.pallas_call(
        _norm_l0_body,
        grid=(B,),
        compiler_params=_cp(1),
        in_specs=[
            pl.BlockSpec((1, 3, N), lambda i: (i, 0, 0)),
            pl.BlockSpec((24, 3), lambda i: (0, 0)),
            pl.BlockSpec((24, 1), lambda i: (0, 0)),
        ],
        out_specs=[
            pl.BlockSpec((1, 3, N), lambda i: (i, 0, 0)),
            pl.BlockSpec((1, 3, 1), lambda i: (i, 0, 0)),
            pl.BlockSpec((1, 1, 1), lambda i: (i, 0, 0)),
            pl.BlockSpec((1, 24, N), lambda i: (i, 0, 0)),
        ],
        out_shape=[
            jax.ShapeDtypeStruct((B, 3, N), F32),
            jax.ShapeDtypeStruct((B, 3, 1), F32),
            jax.ShapeDtypeStruct((B, 1, 1), F32),
            jax.ShapeDtypeStruct((B, 24, N), F32),
        ],
    )(xyz, w, b.reshape(24, 1))


def _ec_body(x_ref, xr_ref, wu_ref, wv_ref, w1a_ref, w1b_ref, w2a_ref,
             w2b_ref, w2c_ref, b0_ref, b1_ref, b2_ref, y_ref, *, T):
    x = x_ref[0]
    xr = xr_ref[0]
    N = x.shape[1]
    base = pl.program_id(1) * T

    pp = jnp.sum(x * x, axis=0, keepdims=True)
    d = pp - 2.0 * _dot(xr, x, ((0,), (0,)))

    lane = jax.lax.broadcasted_iota(jnp.int32, (T, N), 1)
    row = base + jax.lax.broadcasted_iota(jnp.int32, (T, N), 0)
    d = jnp.where(lane == row, BIG, d)

    v0t = _dot(x, wv_ref[...], ((0,), (1,)))
    u0 = _dot(xr, wu_ref[...], ((0,), (1,))) + b0_ref[...]
    w1x = _dot(xr, w1b_ref[...], ((0,), (1,))) + b1_ref[...]
    w2x = _dot(xr, w2c_ref[...], ((0,), (1,))) + b2_ref[...]

    w1a = w1a_ref[...]
    w2a = w2a_ref[...]
    w2b = w2b_ref[...]

    gs = []
    for _ in range(KNN):
        idx = jnp.argmin(d, axis=1).astype(jnp.int32).reshape(T, 1)
        onehot = lane == idx
        sel = onehot.astype(F32)
        gs.append(_dot(sel, v0t, ((1,), (0,))))
        d = jnp.where(onehot, BIG, d)

    g = jnp.concatenate(gs, axis=0)
    r0 = jax.nn.relu(jnp.tile(u0, (KNN, 1)) + g)
    r1 = jax.nn.relu(_dot(r0, w1a, ((1,), (1,))) + jnp.tile(w1x, (KNN, 1)))
    c2 = (_dot(r1, w2a, ((1,), (1,))) + _dot(r0, w2b, ((1,), (1,)))
          + jnp.tile(w2x, (KNN, 1)))
    c2a = jnp.max(c2.reshape(KNN, T, 12), axis=0)
    r1a = jnp.max(r1.reshape(KNN, T, 12), axis=0)
    r0a = jnp.max(r0.reshape(KNN, T, 12), axis=0)

    y = jnp.concatenate([c2a, r1a, r0a], axis=1)
    y_ref[0] = y.T


def _edge_conv(x, p):
    B, _, N = x.shape
    T = _block_t(N)
    nb = _cdiv(N, T)
    w0, w1, w2 = p['W0'], p['W1'], p['W2']
    wu = w0[:, :24] - w0[:, 24:]
    wv = w0[:, 24:]
    args = (x, x, wu, wv, w1[:, :12], w1[:, 12:], w2[:, :12], w2[:, 12:24],
            w2[:, 24:], p['b0'].reshape(1, 12), p['b1'].reshape(1, 12),
            p['b2'].reshape(1, 12))
    wspec = lambda s: pl.BlockSpec(s, lambda b, i: (0,) * len(s))
    return pl.pallas_call(
        functools.partial(_ec_body, T=T),
        grid=(B, nb),
        compiler_params=_cp(2),
        in_specs=[
            pl.BlockSpec((1, 24, N), lambda b, i: (b, 0, 0)),
            pl.BlockSpec((1, 24, T), lambda b, i: (b, 0, i)),
            wspec((12, 24)), wspec((12, 24)), wspec((12, 12)),
            wspec((12, 24)), wspec((12, 12)), wspec((12, 12)),
            wspec((12, 24)), wspec((1, 12)), wspec((1, 12)), wspec((1, 12)),
        ],
        out_specs=pl.BlockSpec((1, 36, T), lambda b, i: (b, 0, i)),
        out_shape=jax.ShapeDtypeStruct((B, 36, N), F32),
    )(*args)


def _proj_body(*refs, nparts):
    parts = refs[:nparts]
    ws = refs[nparts:2 * nparts]
    b_ref = refs[2 * nparts]
    out_ref = refs[2 * nparts + 1]
    acc = b_ref[...].T
    for pr, wr in zip(parts, ws):
        acc = acc + _dot(wr[...], pr[0], ((1,), (0,)))
    out_ref[0] = jax.nn.relu(acc)


def _proj(parts, ws, b):
    B, _, N = parts[0].shape
    T = _block_t(N)
    nb = _cdiv(N, T)
    in_specs = [pl.BlockSpec((1, p.shape[1], T), lambda b_, i: (b_, 0, i))
                for p in parts]
    in_specs += [pl.BlockSpec(w.shape, lambda b_, i: (0, 0)) for w in ws]
    in_specs += [pl.BlockSpec((1, 24), lambda b_, i: (0, 0))]
    return pl.pallas_call(
        functools.partial(_proj_body, nparts=len(parts)),
        grid=(B, nb),
        compiler_params=_cp(2),
        in_specs=in_specs,
        out_specs=pl.BlockSpec((1, 24, T), lambda b_, i: (b_, 0, i)),
        out_shape=jax.ShapeDtypeStruct((B, 24, N), F32),
    )(*parts, *ws, b.reshape(1, 24))


def _fuse_knn_body(q_ref, qr_ref, pxyz_ref, pfeat_ref, x_ref,
                   sd_ref, fd_ref, feats_ref, *, T):
    qr = qr_ref[0]
    pxyz = pxyz_ref[0]
    pfeat = pfeat_ref[0]
    M = pxyz.shape[1]
    del q_ref

    pp = jnp.sum(pxyz * pxyz, axis=0, keepdims=True)
    d = pp - 2.0 * _dot(qr, pxyz, ((0,), (0,)))
    lane = jax.lax.broadcasted_iota(jnp.int32, (T, M), 1)

    qt = qr.T
    xt = x_ref[0].T

    sds, fds = [], []
    for k in range(FM_KNN):
        idx = jnp.argmin(d, axis=1).astype(jnp.int32).reshape(T, 1)
        onehot = lane == idx
        sel = onehot.astype(F32)
        d = jnp.where(onehot, BIG, d)
        feat = _dot(sel, pfeat, ((1,), (1,)))
        pt = _dot(sel, pxyz, ((1,), (1,)))
        sds.append(jnp.sum((qt - pt) ** 2, axis=1, keepdims=True))
        fds.append(jnp.sum((xt - feat) ** 2, axis=1, keepdims=True))
        feats_ref[0, k] = feat
    sd_ref[0] = jnp.concatenate(sds, axis=1)
    fd_ref[0] = jnp.concatenate(fds, axis=1)


def _fuse_h_body(sd_ref, fd_ref, hs_ref, hf_ref):
    hs_ref[0] = jnp.mean(jnp.min(sd_ref[0], axis=1, keepdims=True),
                         axis=0, keepdims=True)
    hf_ref[0] = jnp.mean(jnp.min(fd_ref[0], axis=1, keepdims=True),
                         axis=0, keepdims=True)


def _fuse_apply_body(x_ref, feats_ref, sd_ref, fd_ref, hs_ref, hf_ref,
                     out_ref):
    hs = hs_ref[0]
    hf = hf_ref[0]
    sw = jnp.exp(-sd_ref[0] / (hs * 0.5))
    fw = jnp.exp(-fd_ref[0] / (hf * 0.5))
    w = sw * fw
    w = w / jnp.sum(w + 1e-05, axis=1, keepdims=True)
    corr = (w[:, 0:1] * feats_ref[0, 0] + w[:, 1:2] * feats_ref[0, 1]
            + w[:, 2:3] * feats_ref[0, 2])
    out_ref[0] = x_ref[0] + 0.2 * corr.T


def _fusion(xyz, prev_xyz, prev_feat, x):
    B, _, N = xyz.shape
    M = prev_xyz.shape[2]
    C = x.shape[1]
    T = _block_t(N)
    nb = _cdiv(N, T)
    sd, fd, feats = pl.pallas_call(
        functools.partial(_fuse_knn_body, T=T),
        grid=(B, nb),
        compiler_params=_cp(2),
        in_specs=[
            pl.BlockSpec((1, 3, N), lambda b, i: (b, 0, 0)),
            pl.BlockSpec((1, 3, T), lambda b, i: (b, 0, i)),
            pl.BlockSpec((1, 3, M), lambda b, i: (b, 0, 0)),
            pl.BlockSpec((1, C, M), lambda b, i: (b, 0, 0)),
            pl.BlockSpec((1, C, T), lambda b, i: (b, 0, i)),
        ],
        out_specs=[
            pl.BlockSpec((1, T, FM_KNN), lambda b, i: (b, i, 0)),
            pl.BlockSpec((1, T, FM_KNN), lambda b, i: (b, i, 0)),
            pl.BlockSpec((1, FM_KNN, T, C), lambda b, i: (b, 0, i, 0)),
        ],
        out_shape=[
            jax.ShapeDtypeStruct((B, N, FM_KNN), F32),
            jax.ShapeDtypeStruct((B, N, FM_KNN), F32),
            jax.ShapeDtypeStruct((B, FM_KNN, N, C), F32),
        ],
    )(xyz, xyz, prev_xyz, prev_feat, x)

    hs, hf = pl.pallas_call(
        _fuse_h_body,
        grid=(B,),
        compiler_params=_cp(1),
        in_specs=[
            pl.BlockSpec((1, N, FM_KNN), lambda b: (b, 0, 0)),
            pl.BlockSpec((1, N, FM_KNN), lambda b: (b, 0, 0)),
        ],
        out_specs=[
            pl.BlockSpec((1, 1, 1), lambda b: (b, 0, 0)),
            pl.BlockSpec((1, 1, 1), lambda b: (b, 0, 0)),
        ],
        out_shape=[
            jax.ShapeDtypeStruct((B, 1, 1), F32),
            jax.ShapeDtypeStruct((B, 1, 1), F32),
        ],
    )(sd, fd)

    return pl.pallas_call(
        _fuse_apply_body,
        grid=(B, nb),
        compiler_params=_cp(2),
        in_specs=[
            pl.BlockSpec((1, C, T), lambda b, i: (b, 0, i)),
            pl.BlockSpec((1, FM_KNN, T, C), lambda b, i: (b, 0, i, 0)),
            pl.BlockSpec((1, T, FM_KNN), lambda b, i: (b, i, 0)),
            pl.BlockSpec((1, T, FM_KNN), lambda b, i: (b, i, 0)),
            pl.BlockSpec((1, 1, 1), lambda b, i: (b, 0, 0)),
            pl.BlockSpec((1, 1, 1), lambda b, i: (b, 0, 0)),
        ],
        out_specs=pl.BlockSpec((1, C, T), lambda b, i: (b, 0, i)),
        out_shape=jax.ShapeDtypeStruct((B, C, N), F32),
    )(x, feats, sd, fd, hs, hf)


def _ups_body(x_ref, xn_ref, cent_ref, rad_ref, w1_ref, bia_ref, bib_ref,
              w2_ref, b2_ref, w3_ref, b3_ref, w4_ref, b4_ref,
              o0_ref, o1_ref):
    x = x_ref[0]
    t = _dot(w1_ref[...], x, ((1,), (0,)))
    cent = cent_ref[0]
    rad = rad_ref[0]
    base = xn_ref[0]
    for bias_ref, o_ref in ((bia_ref, o0_ref), (bib_ref, o1_ref)):
        u = jax.nn.relu(t + bias_ref[...])
        u = jax.nn.relu(_dot(w2_ref[...], u, ((1,), (0,))) + b2_ref[...])
        u = jax.nn.relu(_dot(w3_ref[...], u, ((1,), (0,))) + b3_ref[...])
        u = _dot(w4_ref[...], u, ((1,), (0,))) + b4_ref[...]
        o_ref[0] = (u + base) * rad + cent


def _upsample(x, xyz_norm, cent, rad, p):
    B, C, N = x.shape
    T = _block_t(N)
    nb = _cdiv(N, T)
    w1 = p['up1_W'][:, :264]
    wc = p['up1_W'][:, 264:]
    code = jnp.linspace(-0.2, 0.2, STEP_RATIO, dtype=F32)
    bia = (p['up1_b'].reshape(128, 1) + wc * code[0])
    bib = (p['up1_b'].reshape(128, 1) + wc * code[1])
    wspec = lambda s: pl.BlockSpec(s, lambda b, i: (0,) * len(s))
    o0, o1 = pl.pallas_call(
        _ups_body,
        grid=(B, nb),
        compiler_params=_cp(2),
        in_specs=[
            pl.BlockSpec((1, C, T), lambda b, i: (b, 0, i)),
            pl.BlockSpec((1, 3, T), lambda b, i: (b, 0, i)),
            pl.BlockSpec((1, 3, 1), lambda b, i: (b, 0, 0)),
            pl.BlockSpec((1, 1, 1), lambda b, i: (b, 0, 0)),
            wspec((128, 264)), wspec((128, 1)), wspec((128, 1)),
            wspec((128, 128)), wspec((128, 1)),
            wspec((64, 128)), wspec((64, 1)),
            wspec((3, 64)), wspec((3, 1)),
        ],
        out_specs=[
            pl.BlockSpec((1, 3, T), lambda b, i: (b, 0, i)),
            pl.BlockSpec((1, 3, T), lambda b, i: (b, 0, i)),
        ],
        out_shape=[
            jax.ShapeDtypeStruct((B, 3, N), F32),
            jax.ShapeDtypeStruct((B, 3, N), F32),
        ],
    )(x, xyz_norm, cent, rad, w1, bia, bib,
      p['up2_W'], p['up2_b'].reshape(128, 1),
      p['fc1_W'], p['fc1_b'].reshape(64, 1),
      p['fc2_W'], p['fc2_b'].reshape(3, 1))
    return jnp.stack([o0, o1], axis=-1).reshape(B, 3, N * STEP_RATIO)


def _level(xyz, prev, p):
    xyz_norm, cent, rad, x0 = _norm_l0(xyz, p['l0_W'], p['l0_b'])
    e1 = _edge_conv(x0, p['ec1'])
    w = p['p2_W']
    xp2 = _proj([e1, x0], [w[:, :36], w[:, 36:60] + w[:, 60:84]], p['p2_b'])
    e2 = _edge_conv(xp2, p['ec2'])
    w = p['p3_W']
    xp3 = _proj([e2, xp2, e1, x0],
                [w[:, :36], w[:, 36:60], w[:, 60:96], w[:, 96:120] + w[:, 120:144]],
                p['p3_b'])
    e3 = _edge_conv(xp3, p['ec3'])
    w = p['p4_W']
    xp4 = _proj([e3, xp3, e2, xp2, e1, x0],
                [w[:, :36], w[:, 36:60], w[:, 60:96], w[:, 96:120],
                 w[:, 120:156], w[:, 156:180] + w[:, 180:204]],
                p['p4_b'])
    e4 = _edge_conv(xp4, p['ec4'])
    x = jnp.concatenate([e4, xp4, e3, xp3, e2, xp2, e1, x0, x0], axis=1)
    if prev is not None:
        prev_xyz, prev_feat = prev
        x = _fusion(xyz, prev_xyz, prev_feat, x)
    new_xyz = _upsample(x, xyz_norm, cent, rad, p)
    return new_xyz, x


def kernel(xyz, params):
    prev = None
    for l in range(1, NUM_LEVELS + 1):
        new_xyz, feats = _level(xyz, prev, params['level_%d' % l])
        prev = (xyz, feats)
        xyz = new_xyz
    return xyz
